# baseline (speedup 1.0000x reference)
import jax
import jax.numpy as jnp
from jax import lax
from jax.experimental import pallas as pl
from jax.experimental.pallas import tpu as pltpu
from jax.experimental.pallas import tpu_sc as plsc

C = 6
NUM_USERS = 100000
NUM_ITEMS = 100000
HIDDEN = 64
BATCH = 16384

NC, NS, L = 2, 16, 16
NW = NC * NS
BPW = BATCH // NW
NCHUNK = 4
CHUNK = BPW // NCHUNK
NBLK = BPW // L
MH = HIDDEN // L


def _body(u_hbm, i_hbm, p_hbm, q_hbm, w_hbm, r_out, p_out,
          uidx, iidx, uoff, ioff, p_rows, q_rows, r_acc, p_acc,
          w_vmem, sem):
  wid = lax.axis_index("s") * NC + lax.axis_index("c")
  base = wid * BPW

  for j in range(NCHUNK):
    pltpu.sync_copy(u_hbm.at[pl.ds(base + j * CHUNK, CHUNK)], uidx.at[j])
    pltpu.sync_copy(i_hbm.at[pl.ds(base + j * CHUNK, CHUNK)], iidx.at[j])
  pltpu.sync_copy(w_hbm, w_vmem)

  zero = jnp.zeros((L,), jnp.float32)
  for k in range(NBLK):
    r_acc[pl.ds(k * L, L)] = zero
    p_acc[pl.ds(k * L, L)] = zero

  iota = lax.iota(jnp.int32, L)
  perms = [iota ^ 1, iota ^ 2, iota ^ 4, iota ^ 8]
  lane_eq = [iota == l for l in range(L)]

  def _unpack(w):
    wi = lax.bitcast_convert_type(w, jnp.int32)
    lo = lax.bitcast_convert_type(lax.shift_left(wi, 16), jnp.float32)
    hi = lax.bitcast_convert_type(wi & jnp.int32(-65536), jnp.float32)
    return lo, hi

  def comp_body(comp, carry):
    for j in range(NCHUNK):
      for k in range(CHUNK // L):
        sl = pl.ds(k * L, L)
        uoff[j, sl] = lax.shift_right_logical(uidx[j, sl], 2)
        ioff[j, sl] = lax.shift_right_logical(iidx[j, sl], 2)

    cw = [w_vmem[pl.ds(comp * (HIDDEN // 2) + m * L, L)] for m in range(2)]
    cun = [_unpack(w) for w in cw]
    dch = w_vmem[pl.ds(C * (HIDDEN // 2), L)]
    dsplat = dch.at[jnp.full((L,), comp, jnp.int32)].get(
        mode="promise_in_bounds")

    for half in range(2):
      copies = []
      for jj in range(2):
        j = half * 2 + jj
        dst = pl.ds(jj * CHUNK, CHUNK)
        copies.append(pltpu.async_copy(p_hbm.at[comp].at[uoff.at[j]],
                                       p_rows.at[dst], sem))
        copies.append(pltpu.async_copy(q_hbm.at[comp].at[ioff.at[j]],
                                       q_rows.at[dst], sem))
      for cp in copies:
        cp.wait()

      def blk_body(bi, carry2, half=half, cun=cun, dsplat=dsplat):
        j = half * 2 + bi // 8
        col = (bi % 8) * L
        u_chunk = uidx[j, pl.ds(col, L)]
        i_chunk = iidx[j, pl.ds(col, L)]
        racc = zero
        pacc = dsplat
        for l in range(L):
          row = bi * L + l
          uo = lax.shift_left(u_chunk[l] & 3, 5)
          io = lax.shift_left(i_chunk[l] & 3, 5)
          t = None
          s = None
          for m in range(2):
            plo, phi = _unpack(p_rows[row, pl.ds(uo + m * L, L)])
            qlo, qhi = _unpack(q_rows[row, pl.ds(io + m * L, L)])
            clo, chi = cun[m]
            tm = plo * qlo + phi * qhi
            sm = qlo * clo + qhi * chi
            t = tm if t is None else t + tm
            s = sm if s is None else s + sm
          for p in perms:
            t = t + t.at[p].get(mode="promise_in_bounds")
            s = s + s.at[p].get(mode="promise_in_bounds")
          racc = jnp.where(lane_eq[l], t, racc)
          pacc = jnp.where(lane_eq[l], s + pacc, pacc)
        pop = 1.0 / (1.0 + jnp.exp(-pacc))
        sl = pl.ds(half * (BPW // 2) + bi * L, L)
        r_acc[sl] = r_acc[sl] + racc
        p_acc[sl] = p_acc[sl] + pop
        return carry2

      lax.fori_loop(0, NBLK // 2, blk_body, None)
    return carry

  lax.fori_loop(0, C, comp_body, None)

  inv = jnp.float32(1.0 / C)
  for k in range(NBLK):
    sl = pl.ds(k * L, L)
    r_acc[sl] = r_acc[sl] * inv
    p_acc[sl] = p_acc[sl] * inv

  pltpu.sync_copy(r_acc, r_out.at[pl.ds(base, BPW)])
  pltpu.sync_copy(p_acc, p_out.at[pl.ds(base, BPW)])


def _quad_view(x):
  xw = lax.bitcast_convert_type(
      x.astype(jnp.bfloat16).reshape(C, NUM_USERS, HIDDEN // 2, 2),
      jnp.float32)
  return xw.reshape(C, NUM_USERS // 4, 128)


@jax.jit
def _run(u_batch, i_batch, p_tab, q_tab, w_flat):
  mesh = plsc.VectorSubcoreMesh(core_axis_name="c", subcore_axis_name="s",
                                num_cores=NC, num_subcores=NS)
  f = pl.kernel(
      _body,
      out_type=[jax.ShapeDtypeStruct((BATCH,), jnp.float32),
                jax.ShapeDtypeStruct((BATCH,), jnp.float32)],
      mesh=mesh,
      compiler_params=pltpu.CompilerParams(needs_layout_passes=False,
                                           use_tc_tiling_on_sc=True),
      scratch_types=[
          pltpu.VMEM((NCHUNK, CHUNK), jnp.int32),
          pltpu.VMEM((NCHUNK, CHUNK), jnp.int32),
          pltpu.VMEM((NCHUNK, CHUNK), jnp.int32),
          pltpu.VMEM((NCHUNK, CHUNK), jnp.int32),
          pltpu.VMEM((BPW // 2, 2 * HIDDEN), jnp.float32),
          pltpu.VMEM((BPW // 2, 2 * HIDDEN), jnp.float32),
          pltpu.VMEM((BPW,), jnp.float32),
          pltpu.VMEM((BPW,), jnp.float32),
          pltpu.VMEM((512,), jnp.float32),
          pltpu.SemaphoreType.DMA,
      ],
  )
  return f(u_batch, i_batch, p_tab, q_tab, w_flat)


def kernel(u_batch, i_batch, P, Q, c, d):
  c_words = lax.bitcast_convert_type(
      c.astype(jnp.bfloat16).reshape(C, HIDDEN // 2, 2),
      jnp.float32).reshape(C * (HIDDEN // 2))
  w_flat = jnp.concatenate(
      [c_words, d.reshape(C),
       jnp.zeros((512 - C * (HIDDEN // 2) - C,), jnp.float32)])
  r, p = _run(u_batch, i_batch, _quad_view(P), _quad_view(Q), w_flat)
  return (r.reshape(-1, 1), p.reshape(-1, 1))


# speedup vs baseline: 2.5108x; 2.5108x over previous
import jax
import jax.numpy as jnp
from jax import lax
from jax.experimental import pallas as pl
from jax.experimental.pallas import tpu as pltpu
from jax.experimental.pallas import tpu_sc as plsc

C = 6
NUM_USERS = 100000
NUM_ITEMS = 100000
HIDDEN = 64
BATCH = 16384

NC, NS, L = 2, 16, 16
NW = NC * NS
BPW = BATCH // NW
NCHUNK = 4
CHUNK = BPW // NCHUNK
NBLK = BPW // L
MH = HIDDEN // L


def _body(u_hbm, i_hbm, p_hbm, q_hbm, w_hbm, r_out, p_out,
          uidx, iidx, uoff, ioff, p_rows, q_rows, r_acc, p_acc,
          w_vmem, sem):
  wid = lax.axis_index("s") * NC + lax.axis_index("c")
  base = wid * BPW

  for j in range(NCHUNK):
    pltpu.sync_copy(u_hbm.at[pl.ds(base + j * CHUNK, CHUNK)], uidx.at[j])
    pltpu.sync_copy(i_hbm.at[pl.ds(base + j * CHUNK, CHUNK)], iidx.at[j])
  pltpu.sync_copy(w_hbm, w_vmem)

  zero = jnp.zeros((L,), jnp.float32)
  for k in range(NBLK):
    r_acc[pl.ds(k * L, L)] = zero
    p_acc[pl.ds(k * L, L)] = zero

  iota = lax.iota(jnp.int32, L)
  perms = [iota ^ 1, iota ^ 2, iota ^ 4, iota ^ 8]
  lane_eq = [iota == l for l in range(L)]

  for j in range(NCHUNK):
    for k in range(CHUNK // L):
      sl = pl.ds(k * L, L)
      uoff[j, sl] = lax.shift_right_logical(uidx[j, sl], 1)
      ioff[j, sl] = lax.shift_right_logical(iidx[j, sl], 1)

  NSTEP = C * NCHUNK

  pltpu.async_copy(p_hbm.at[0].at[uoff.at[0]], p_rows.at[0], sem)
  pltpu.async_copy(q_hbm.at[0].at[ioff.at[0]], q_rows.at[0], sem)

  def step_body(st, carry):
    comp = lax.shift_right_logical(st, 2)
    q = st & 3
    jb = st & 1
    pltpu.make_async_copy(p_hbm.at[comp].at[uoff.at[q]],
                          p_rows.at[jb], sem).wait()
    pltpu.make_async_copy(q_hbm.at[comp].at[ioff.at[q]],
                          q_rows.at[jb], sem).wait()

    @pl.when(st < NSTEP - 1)
    def _prefetch():
      nst = st + 1
      ncomp = lax.shift_right_logical(nst, 2)
      nq = nst & 3
      njb = nst & 1
      pltpu.async_copy(p_hbm.at[ncomp].at[uoff.at[nq]], p_rows.at[njb], sem)
      pltpu.async_copy(q_hbm.at[ncomp].at[ioff.at[nq]], q_rows.at[njb], sem)

    cch = [w_vmem[pl.ds(comp * HIDDEN + m * L, L)] for m in range(MH)]
    dch = w_vmem[pl.ds(C * HIDDEN, L)]
    dsplat = dch.at[jnp.full((L,), comp, jnp.int32)].get(
        mode="promise_in_bounds")

    def blk_body(bi, carry2):
      col = bi * L
      u_chunk = uidx[q, pl.ds(col, L)]
      i_chunk = iidx[q, pl.ds(col, L)]
      racc = zero
      pacc = dsplat
      for l in range(L):
        row = bi * L + l
        uo = lax.shift_left(u_chunk[l] & 1, 6)
        io = lax.shift_left(i_chunk[l] & 1, 6)
        t = None
        s = None
        for m in range(MH):
          pv = p_rows[jb, row, pl.ds(uo + m * L, L)]
          qv = q_rows[jb, row, pl.ds(io + m * L, L)]
          t = pv * qv if t is None else t + pv * qv
          s = qv * cch[m] if s is None else s + qv * cch[m]
        for p in perms:
          t = t + t.at[p].get(mode="promise_in_bounds")
          s = s + s.at[p].get(mode="promise_in_bounds")
        racc = jnp.where(lane_eq[l], t, racc)
        pacc = jnp.where(lane_eq[l], s + pacc, pacc)
      pop = 1.0 / (1.0 + jnp.exp(-pacc))
      sl = pl.ds(q * CHUNK + bi * L, L)
      r_acc[sl] = r_acc[sl] + racc
      p_acc[sl] = p_acc[sl] + pop
      return carry2

    lax.fori_loop(0, CHUNK // L, blk_body, None)
    return carry

  lax.fori_loop(0, NSTEP, step_body, None)

  inv = jnp.float32(1.0 / C)
  for k in range(NBLK):
    sl = pl.ds(k * L, L)
    r_acc[sl] = r_acc[sl] * inv
    p_acc[sl] = p_acc[sl] * inv

  pltpu.sync_copy(r_acc, r_out.at[pl.ds(base, BPW)])
  pltpu.sync_copy(p_acc, p_out.at[pl.ds(base, BPW)])


def _pair_view(x):
  return x.reshape(C, NUM_USERS // 2, 2 * HIDDEN)


@jax.jit
def _run(u_batch, i_batch, p_tab, q_tab, w_flat):
  mesh = plsc.VectorSubcoreMesh(core_axis_name="c", subcore_axis_name="s",
                                num_cores=NC, num_subcores=NS)
  f = pl.kernel(
      _body,
      out_type=[jax.ShapeDtypeStruct((BATCH,), jnp.float32),
                jax.ShapeDtypeStruct((BATCH,), jnp.float32)],
      mesh=mesh,
      compiler_params=pltpu.CompilerParams(needs_layout_passes=False,
                                           use_tc_tiling_on_sc=True),
      scratch_types=[
          pltpu.VMEM((NCHUNK, CHUNK), jnp.int32),
          pltpu.VMEM((NCHUNK, CHUNK), jnp.int32),
          pltpu.VMEM((NCHUNK, CHUNK), jnp.int32),
          pltpu.VMEM((NCHUNK, CHUNK), jnp.int32),
          pltpu.VMEM((2, CHUNK, 2 * HIDDEN), jnp.float32),
          pltpu.VMEM((2, CHUNK, 2 * HIDDEN), jnp.float32),
          pltpu.VMEM((BPW,), jnp.float32),
          pltpu.VMEM((BPW,), jnp.float32),
          pltpu.VMEM((512,), jnp.float32),
          pltpu.SemaphoreType.DMA,
      ],
  )
  return f(u_batch, i_batch, p_tab, q_tab, w_flat)


def kernel(u_batch, i_batch, P, Q, c, d):
  w_flat = jnp.concatenate(
      [c.reshape(C * HIDDEN), d.reshape(C),
       jnp.zeros((512 - C * HIDDEN - C,), jnp.float32)])
  r, p = _run(u_batch, i_batch, _pair_view(P), _pair_view(Q), w_flat)
  return (r.reshape(-1, 1), p.reshape(-1, 1))
